# baseline (speedup 1.0000x reference)
import functools

import jax
import jax.numpy as jnp
from jax.experimental import pallas as pl

_B, _N, _D, _K = 32, 1024, 64, 1024
_ROWS = 1024


def _vq_body(x_ref, e_ref, idx_ref, q_ref):
    e = e_ref[...]
    en = e / jnp.clip(
        jnp.sqrt(jnp.sum(e * e, axis=1, keepdims=True)), 1e-12)
    xt = x_ref[...]
    xn = xt / jnp.clip(
        jnp.sqrt(jnp.sum(xt * xt, axis=1, keepdims=True)), 1e-12)
    dist = jax.lax.dot_general(
        xn, en, (((1,), (1,)), ((), ())),
        preferred_element_type=jnp.float32)
    m = jnp.max(dist, axis=1, keepdims=True)
    ids = jax.lax.broadcasted_iota(jnp.int32, dist.shape, 1)
    idx = jnp.min(jnp.where(dist == m, ids, _K), axis=1)
    idx_ref[0, 0, :] = idx
    onehot = (ids == idx[:, None]).astype(jnp.float32)
    q_ref[...] = jax.lax.dot_general(
        onehot, e, (((1,), (0,)), ((), ())),
        preferred_element_type=jnp.float32,
        precision=jax.lax.Precision.HIGHEST)


@functools.partial(jax.jit, static_argnums=())
def kernel(x, embed):
    xf = x.reshape(_B * _N, _D)
    e2 = embed[0]
    grid = (_B * _N) // _ROWS
    idx_out, q_out = pl.pallas_call(
        _vq_body,
        grid=(grid,),
        in_specs=[
            pl.BlockSpec((_ROWS, _D), lambda i: (i, 0)),
            pl.BlockSpec((_K, _D), lambda i: (0, 0)),
        ],
        out_specs=[
            pl.BlockSpec((1, 1, _ROWS), lambda i: (i, 0, 0)),
            pl.BlockSpec((_ROWS, _D), lambda i: (i, 0)),
        ],
        out_shape=[
            jax.ShapeDtypeStruct((grid, 1, _ROWS), jnp.int32),
            jax.ShapeDtypeStruct((_B * _N, _D), jnp.float32),
        ],
    )(xf, e2)
    return q_out.reshape(_B, _N, _D), idx_out.reshape(_B, _N)


# speedup vs baseline: 1.0158x; 1.0158x over previous
import jax
import jax.numpy as jnp
from jax.experimental import pallas as pl
from jax.experimental.pallas import tpu as pltpu

_B, _N, _D, _K = 32, 1024, 64, 1024
_ROWS = 1024


def _vq_body(x_ref, e_ref, idx_ref, q_ref, en_ref, wc_ref):
    @pl.when(pl.program_id(0) == 0)
    def _init():
        e = e_ref[...]
        en_ref[...] = e / jnp.clip(
            jnp.sqrt(jnp.sum(e * e, axis=1, keepdims=True)), 1e-12)
        col = jax.lax.broadcasted_iota(jnp.int32, (_K, 64), 1)
        kio = jax.lax.broadcasted_iota(jnp.int32, (_K, 64), 0).astype(jnp.float32)
        extra = jnp.where(col == 0, kio, jnp.where(col == 1, 1.0, 0.0))
        wc_ref[...] = jnp.concatenate([e, extra], axis=1)

    xt = x_ref[...]
    xn = xt / jnp.clip(
        jnp.sqrt(jnp.sum(xt * xt, axis=1, keepdims=True)), 1e-12)
    dist = jax.lax.dot_general(
        xn, en_ref[...], (((1,), (1,)), ((), ())),
        preferred_element_type=jnp.float32)
    m = jnp.max(dist, axis=1, keepdims=True)
    eqf = jnp.where(dist == m, 1.0, 0.0)
    sums = jax.lax.dot_general(
        eqf, wc_ref[...], (((1,), (0,)), ((), ())),
        preferred_element_type=jnp.float32,
        precision=jax.lax.Precision.HIGHEST)
    tie = jnp.max(sums[:, 65]) > 1.5

    @pl.when(jnp.logical_not(tie))
    def _fast():
        idx_ref[0, 0, :] = sums[:, 64].astype(jnp.int32)
        q_ref[...] = sums[:, :64]

    @pl.when(tie)
    def _slow():
        ids = jax.lax.broadcasted_iota(jnp.int32, dist.shape, 1)
        idxt = jnp.min(jnp.where(dist == m, ids, _K), axis=1)
        idx_ref[0, 0, :] = idxt
        oh = (ids == idxt[:, None]).astype(jnp.float32)
        q_ref[...] = jax.lax.dot_general(
            oh, e_ref[...], (((1,), (0,)), ((), ())),
            preferred_element_type=jnp.float32,
            precision=jax.lax.Precision.HIGHEST)


def kernel(x, embed):
    xf = x.reshape(_B * _N, _D)
    e2 = embed[0]
    grid = (_B * _N) // _ROWS
    idx_out, q_out = pl.pallas_call(
        _vq_body,
        grid=(grid,),
        in_specs=[
            pl.BlockSpec((_ROWS, _D), lambda i: (i, 0)),
            pl.BlockSpec((_K, _D), lambda i: (0, 0)),
        ],
        out_specs=[
            pl.BlockSpec((1, 1, _ROWS), lambda i: (i, 0, 0)),
            pl.BlockSpec((_ROWS, _D), lambda i: (i, 0)),
        ],
        out_shape=[
            jax.ShapeDtypeStruct((grid, 1, _ROWS), jnp.int32),
            jax.ShapeDtypeStruct((_B * _N, _D), jnp.float32),
        ],
        scratch_shapes=[
            pltpu.VMEM((_K, _D), jnp.float32),
            pltpu.VMEM((_K, 128), jnp.float32),
        ],
    )(xf, e2)
    return q_out.reshape(_B, _N, _D), idx_out.reshape(_B, _N)


# speedup vs baseline: 1.7919x; 1.7640x over previous
import jax
import jax.numpy as jnp
from jax.experimental import pallas as pl
from jax.experimental.pallas import tpu as pltpu

_B, _N, _D, _K = 32, 1024, 64, 1024
_BB = 8
_GRID = _B // _BB


def _vq_body(x_ref, e_ref, idx_ref, q_ref, en_ref, wc_ref):
    @pl.when(pl.program_id(0) == 0)
    def _init():
        e = e_ref[...]
        en_ref[...] = e / jnp.clip(
            jnp.sqrt(jnp.sum(e * e, axis=1, keepdims=True)), 1e-12)
        col = jax.lax.broadcasted_iota(jnp.int32, (_K, 64), 1)
        kio = jax.lax.broadcasted_iota(jnp.int32, (_K, 64), 0).astype(
            jnp.float32)
        extra = jnp.where(col == 0, kio, jnp.where(col == 1, 1.0, 0.0))
        wc_ref[...] = jnp.concatenate([e_ref[...], extra], axis=1)

    def _tile(j, carry):
        xt = x_ref[j]
        xn = xt / jnp.clip(
            jnp.sqrt(jnp.sum(xt * xt, axis=1, keepdims=True)), 1e-12)
        dist = jax.lax.dot_general(
            xn, en_ref[...], (((1,), (1,)), ((), ())),
            preferred_element_type=jnp.float32)
        m = jnp.max(dist, axis=1, keepdims=True)
        eqf = jnp.where(dist == m, 1.0, 0.0)
        sums = jax.lax.dot_general(
            eqf, wc_ref[...], (((1,), (0,)), ((), ())),
            preferred_element_type=jnp.float32)
        tie = jnp.max(sums[:, 65]) > 1.5

        @pl.when(jnp.logical_not(tie))
        def _fast():
            idx_ref[pl.ds(j, 1), :] = sums[:, 64].astype(jnp.int32)[None, :]
            q_ref[j] = sums[:, :64]

        @pl.when(tie)
        def _slow():
            ids = jax.lax.broadcasted_iota(jnp.int32, dist.shape, 1)
            idxt = jnp.min(jnp.where(dist == m, ids, _K), axis=1)
            idx_ref[pl.ds(j, 1), :] = idxt[None, :]
            oh = (ids == idxt[:, None]).astype(jnp.float32)
            q_ref[j] = jax.lax.dot_general(
                oh, e_ref[...], (((1,), (0,)), ((), ())),
                preferred_element_type=jnp.float32,
                precision=jax.lax.Precision.HIGHEST)

        return carry

    jax.lax.fori_loop(0, _BB, _tile, 0)


def kernel(x, embed):
    e2 = embed[0]
    idx_out, q_out = pl.pallas_call(
        _vq_body,
        grid=(_GRID,),
        in_specs=[
            pl.BlockSpec((_BB, _N, _D), lambda i: (i, 0, 0)),
            pl.BlockSpec((_K, _D), lambda i: (0, 0)),
        ],
        out_specs=[
            pl.BlockSpec((_BB, _N), lambda i: (i, 0)),
            pl.BlockSpec((_BB, _N, _D), lambda i: (i, 0, 0)),
        ],
        out_shape=[
            jax.ShapeDtypeStruct((_B, _N), jnp.int32),
            jax.ShapeDtypeStruct((_B, _N, _D), jnp.float32),
        ],
        scratch_shapes=[
            pltpu.VMEM((_K, _D), jnp.float32),
            pltpu.VMEM((_K, 128), jnp.float32),
        ],
    )(x, e2)
    return q_out, idx_out


# speedup vs baseline: 1.9391x; 1.0821x over previous
import jax
import jax.numpy as jnp
from jax.experimental import pallas as pl
from jax.experimental.pallas import tpu as pltpu

_B, _N, _D, _K = 32, 1024, 64, 1024
_BB = 8
_GRID = _B // _BB


def _vq_body(x_ref, e_ref, idx_ref, q_ref, en_ref, wc_ref):
    @pl.when(pl.program_id(0) == 0)
    def _init():
        e = e_ref[...]
        en_ref[...] = e / jnp.clip(
            jnp.sqrt(jnp.sum(e * e, axis=1, keepdims=True)), 1e-12)
        col = jax.lax.broadcasted_iota(jnp.int32, (_K, 64), 1)
        kio = jax.lax.broadcasted_iota(jnp.int32, (_K, 64), 0).astype(
            jnp.float32)
        extra = jnp.where(col == 0, kio, jnp.where(col == 1, 1.0, 0.0))
        wc_ref[...] = jnp.concatenate([e_ref[...], extra], axis=1)

    def _tile(j, carry):
        xt = x_ref[pl.ds(2 * j, 2)].reshape(2 * _N, _D)
        xn = xt / jnp.clip(
            jnp.sqrt(jnp.sum(xt * xt, axis=1, keepdims=True)), 1e-12)
        dist = jax.lax.dot_general(
            xn, en_ref[...], (((1,), (1,)), ((), ())),
            preferred_element_type=jnp.float32)
        m = jnp.max(dist, axis=1, keepdims=True)
        eqf = jnp.where(dist == m, 1.0, 0.0)
        sums = jax.lax.dot_general(
            eqf, wc_ref[...], (((1,), (0,)), ((), ())),
            preferred_element_type=jnp.float32)
        tie = jnp.max(sums[:, 65]) > 1.5

        def _store(idx2d, q3d):
            idx_ref[pl.ds(2 * j, 1), :] = idx2d[0:1]
            idx_ref[pl.ds(2 * j + 1, 1), :] = idx2d[1:2]
            q_ref[pl.ds(2 * j, 1)] = q3d[0:1]
            q_ref[pl.ds(2 * j + 1, 1)] = q3d[1:2]

        @pl.when(jnp.logical_not(tie))
        def _fast():
            _store(sums[:, 64].astype(jnp.int32).reshape(2, _N),
                   sums[:, :64].reshape(2, _N, _D))

        @pl.when(tie)
        def _slow():
            ids = jax.lax.broadcasted_iota(jnp.int32, dist.shape, 1)
            idxt = jnp.min(jnp.where(dist == m, ids, _K), axis=1)
            oh = (ids == idxt[:, None]).astype(jnp.float32)
            qv = jax.lax.dot_general(
                oh, e_ref[...], (((1,), (0,)), ((), ())),
                preferred_element_type=jnp.float32,
                precision=jax.lax.Precision.HIGHEST)
            _store(idxt.reshape(2, _N), qv.reshape(2, _N, _D))

        return carry

    jax.lax.fori_loop(0, _BB // 2, _tile, 0)


def kernel(x, embed):
    e2 = embed[0]
    idx_out, q_out = pl.pallas_call(
        _vq_body,
        grid=(_GRID,),
        in_specs=[
            pl.BlockSpec((_BB, _N, _D), lambda i: (i, 0, 0)),
            pl.BlockSpec((_K, _D), lambda i: (0, 0)),
        ],
        out_specs=[
            pl.BlockSpec((_BB, _N), lambda i: (i, 0)),
            pl.BlockSpec((_BB, _N, _D), lambda i: (i, 0, 0)),
        ],
        out_shape=[
            jax.ShapeDtypeStruct((_B, _N), jnp.int32),
            jax.ShapeDtypeStruct((_B, _N, _D), jnp.float32),
        ],
        scratch_shapes=[
            pltpu.VMEM((_K, _D), jnp.float32),
            pltpu.VMEM((_K, 128), jnp.float32),
        ],
    )(x, e2)
    return q_out, idx_out
